# baseline (speedup 1.0000x reference)
import functools

import jax
import jax.numpy as jnp
from jax import lax
from jax.experimental import pallas as pl
from jax.experimental.pallas import tpu as pltpu
from jax.experimental.pallas import tpu_sc as plsc

BATCH = 16384
NFEAT = 4096
NCOLS = 64

NCORES = 2
NSUB = 16
NWORKERS = NCORES * NSUB
ROWS_PER_W = BATCH // NWORKERS
CHUNK = 128
ROWS_PER_CHUNK = CHUNK // NCOLS
NCHUNKS = ROWS_PER_W // ROWS_PER_CHUNK
WAVE = 8
LANES = 16
VPC = CHUNK // LANES


@functools.partial(
    pl.kernel,
    out_type=jax.ShapeDtypeStruct((BATCH * NCOLS,), jnp.float32),
    mesh=plsc.VectorSubcoreMesh(core_axis_name="c", subcore_axis_name="s"),
    scratch_types=[
        pltpu.VMEM((NCOLS,), jnp.int32),
        pltpu.VMEM((NCHUNKS, CHUNK), jnp.int32),
        pltpu.VMEM((ROWS_PER_W * NCOLS,), jnp.float32),
        pltpu.SemaphoreType.DMA,
    ],
)
def _sc_gather_cols(x_hbm, cols_hbm, out_hbm, colv, idxv, gv, sem):
    wid = lax.axis_index("s") * NCORES + lax.axis_index("c")
    row0 = wid * ROWS_PER_W

    pltpu.sync_copy(cols_hbm, colv)

    vs0 = tuple(
        colv[pl.ds((s % (NCOLS // LANES)) * LANES, LANES)]
        + (row0 + s // (NCOLS // LANES)) * NFEAT
        for s in range(VPC)
    )

    def build(n, vs):
        for s in range(VPC):
            idxv[n, pl.ds(s * LANES, LANES)] = vs[s]
        return tuple(v + ROWS_PER_CHUNK * NFEAT for v in vs)

    lax.fori_loop(0, NCHUNKS, build, vs0)

    def wave(w, carry):
        copies = []
        for k in range(WAVE):
            n = w * WAVE + k
            copies.append(
                pltpu.async_copy(
                    x_hbm.at[idxv.at[n]],
                    gv.at[pl.ds(n * CHUNK, CHUNK)],
                    sem,
                )
            )
        for c in copies:
            c.wait()
        return carry

    lax.fori_loop(0, NCHUNKS // WAVE, wave, 0)

    pltpu.sync_copy(gv, out_hbm.at[pl.ds(row0 * NCOLS, ROWS_PER_W * NCOLS)])


def kernel(x, column_indices):
    out = _sc_gather_cols(x.reshape(-1), column_indices)
    return out.reshape(BATCH, NCOLS)


# speedup vs baseline: 2.9497x; 2.9497x over previous
import functools

import jax
import jax.numpy as jnp
from jax import lax
from jax.experimental import pallas as pl
from jax.experimental.pallas import tpu as pltpu
from jax.experimental.pallas import tpu_sc as plsc

BATCH = 16384
NFEAT = 4096
NCOLS = 64

NCORES = 2
NSUB = 16
NWORKERS = NCORES * NSUB
ROWS_PER_W = BATCH // NWORKERS
CHUNK = 128
ROWS_PER_CHUNK = CHUNK // NCOLS
NCHUNKS = ROWS_PER_W // ROWS_PER_CHUNK
WAVE = 8
LANES = 16
VPC = CHUNK // LANES
CVR = NCOLS // LANES

SUB = 8
LANES_TC = 128
TILE_WORDS = SUB * LANES_TC
TROW_WORDS = (NFEAT // LANES_TC) * TILE_WORDS
TROWS_PER_W = ROWS_PER_W // SUB
CHUNKS_PER_TROW = SUB // ROWS_PER_CHUNK


@functools.partial(
    pl.kernel,
    out_type=jax.ShapeDtypeStruct((BATCH * NCOLS,), jnp.float32),
    mesh=plsc.VectorSubcoreMesh(core_axis_name="c", subcore_axis_name="s"),
    scratch_types=[
        pltpu.VMEM((NCOLS,), jnp.int32),
        pltpu.VMEM((NCHUNKS, CHUNK), jnp.int32),
        pltpu.VMEM((ROWS_PER_W * NCOLS,), jnp.float32),
        pltpu.SemaphoreType.DMA,
    ],
)
def _sc_gather_cols(x_hbm, cols_hbm, out_hbm, colv, idxv, gv, sem):
    wid = lax.axis_index("s") * NCORES + lax.axis_index("c")
    row0 = wid * ROWS_PER_W
    trow0 = row0 // SUB

    pltpu.sync_copy(cols_hbm, colv)

    cts = []
    for m in range(CVR):
        c = colv[pl.ds(m * LANES, LANES)]
        cts.append(c + lax.shift_right_logical(c, 7) * (TILE_WORDS - LANES_TC))
    cts = tuple(cts)

    def build(t, ct):
        stripe = (trow0 + t) * TROW_WORDS
        for k in range(CHUNKS_PER_TROW):
            n = t * CHUNKS_PER_TROW + k
            for s in range(VPC):
                sub = 2 * k + s // CVR
                idxv[n, pl.ds(s * LANES, LANES)] = (
                    ct[s % CVR] + (stripe + sub * LANES_TC)
                )
        return ct

    lax.fori_loop(0, TROWS_PER_W, build, cts)

    def wave(w, carry):
        copies = []
        for k in range(WAVE):
            n = w * WAVE + k
            copies.append(
                pltpu.async_copy(
                    x_hbm.at[idxv.at[n]],
                    gv.at[pl.ds(n * CHUNK, CHUNK)],
                    sem,
                )
            )
        for c in copies:
            c.wait()
        return carry

    lax.fori_loop(0, NCHUNKS // WAVE, wave, 0)

    pltpu.sync_copy(gv, out_hbm.at[pl.ds(row0 * NCOLS, ROWS_PER_W * NCOLS)])


def kernel(x, column_indices):
    x_tiles = (
        x.reshape(BATCH // SUB, SUB, NFEAT // LANES_TC, LANES_TC)
        .swapaxes(1, 2)
        .reshape(-1)
    )
    out = _sc_gather_cols(x_tiles, column_indices)
    return out.reshape(BATCH, NCOLS)


# speedup vs baseline: 3.6971x; 1.2534x over previous
import functools

import jax
import jax.numpy as jnp
from jax import lax
from jax.experimental import pallas as pl
from jax.experimental.pallas import tpu as pltpu
from jax.experimental.pallas import tpu_sc as plsc

BATCH = 16384
NFEAT = 4096
NCOLS = 64

NCORES = 2
NSUB = 16
NWORKERS = NCORES * NSUB
ROWS_PER_W = BATCH // NWORKERS
CHUNK = 128
ROWS_PER_CHUNK = CHUNK // NCOLS
NCHUNKS = ROWS_PER_W // ROWS_PER_CHUNK
LANES = 16
VPC = CHUNK // LANES
CVR = NCOLS // LANES

SUB = 8
LANES_TC = 128
TILE_WORDS = SUB * LANES_TC
TROW_WORDS = (NFEAT // LANES_TC) * TILE_WORDS
TROWS_PER_W = ROWS_PER_W // SUB
CHUNKS_PER_TROW = SUB // ROWS_PER_CHUNK


@functools.partial(
    pl.kernel,
    out_type=jax.ShapeDtypeStruct((BATCH * NCOLS,), jnp.float32),
    mesh=plsc.VectorSubcoreMesh(core_axis_name="c", subcore_axis_name="s"),
    scratch_types=[
        pltpu.VMEM((NCOLS,), jnp.int32),
        pltpu.VMEM((NCHUNKS, CHUNK), jnp.int32),
        pltpu.VMEM((ROWS_PER_W * NCOLS,), jnp.float32),
        pltpu.SemaphoreType.DMA,
    ],
)
def _sc_gather_cols(x_hbm, cols_hbm, out_hbm, colv, idxv, gv, sem):
    wid = lax.axis_index("s") * NCORES + lax.axis_index("c")
    row0 = wid * ROWS_PER_W
    trow0 = row0 // SUB

    pltpu.sync_copy(cols_hbm, colv)

    cts = []
    for m in range(CVR):
        c = colv[pl.ds(m * LANES, LANES)]
        cts.append(c + lax.shift_right_logical(c, 7) * (TILE_WORDS - LANES_TC))
    cts = tuple(cts)

    def build(t, ct):
        stripe = (trow0 + t) * TROW_WORDS
        for k in range(CHUNKS_PER_TROW):
            n = t * CHUNKS_PER_TROW + k
            for s in range(VPC):
                sub = 2 * k + s // CVR
                idxv[n, pl.ds(s * LANES, LANES)] = (
                    ct[s % CVR] + (stripe + sub * LANES_TC)
                )
        return ct

    lax.fori_loop(0, TROWS_PER_W, build, cts)

    def fire(n, carry):
        pltpu.async_copy(
            x_hbm.at[idxv.at[n]], gv.at[pl.ds(n * CHUNK, CHUNK)], sem
        )
        return carry

    lax.fori_loop(0, NCHUNKS, fire, 0)

    def drain(n, carry):
        pltpu.make_async_copy(
            x_hbm.at[idxv.at[n]], gv.at[pl.ds(n * CHUNK, CHUNK)], sem
        ).wait()
        return carry

    lax.fori_loop(0, NCHUNKS, drain, 0)

    pltpu.sync_copy(gv, out_hbm.at[pl.ds(row0 * NCOLS, ROWS_PER_W * NCOLS)])


def kernel(x, column_indices):
    x_tiles = (
        x.reshape(BATCH // SUB, SUB, NFEAT // LANES_TC, LANES_TC)
        .swapaxes(1, 2)
        .reshape(-1)
    )
    out = _sc_gather_cols(x_tiles, column_indices)
    return out.reshape(BATCH, NCOLS)


# speedup vs baseline: 3.7604x; 1.0171x over previous
import functools

import jax
import jax.numpy as jnp
from jax import lax
from jax.experimental import pallas as pl
from jax.experimental.pallas import tpu as pltpu
from jax.experimental.pallas import tpu_sc as plsc

BATCH = 16384
NFEAT = 4096
NCOLS = 64

NCORES = 2
NSUB = 16
NWORKERS = NCORES * NSUB
ROWS_PER_W = BATCH // NWORKERS
CHUNK = 128
ROWS_PER_CHUNK = CHUNK // NCOLS
NCHUNKS = ROWS_PER_W // ROWS_PER_CHUNK
LANES = 16
VPC = CHUNK // LANES
CVR = NCOLS // LANES

SUB = 8
LANES_TC = 128
TILE_WORDS = SUB * LANES_TC
TROW_WORDS = (NFEAT // LANES_TC) * TILE_WORDS
TROWS_PER_W = ROWS_PER_W // SUB
CHUNKS_PER_TROW = SUB // ROWS_PER_CHUNK


@functools.partial(
    pl.kernel,
    out_type=jax.ShapeDtypeStruct((BATCH * NCOLS,), jnp.float32),
    mesh=plsc.VectorSubcoreMesh(core_axis_name="c", subcore_axis_name="s"),
    scratch_types=[
        pltpu.VMEM((NCOLS,), jnp.int32),
        pltpu.VMEM((NCHUNKS, CHUNK), jnp.int32),
        pltpu.VMEM((ROWS_PER_W * NCOLS,), jnp.float32),
        pltpu.SemaphoreType.DMA,
    ],
)
def _sc_gather_cols(x_hbm, cols_hbm, out_hbm, colv, idxv, gv, sem):
    wid = lax.axis_index("s") * NCORES + lax.axis_index("c")
    row0 = wid * ROWS_PER_W
    trow0 = row0 // SUB

    pltpu.sync_copy(cols_hbm, colv)

    cts = []
    for m in range(CVR):
        c = colv[pl.ds(m * LANES, LANES)]
        cts.append(c + lax.shift_right_logical(c, 7) * (TILE_WORDS - LANES_TC))
    cts = tuple(cts)

    def build_fire(t, ct):
        stripe = (trow0 + t) * TROW_WORDS
        for k in range(CHUNKS_PER_TROW):
            n = t * CHUNKS_PER_TROW + k
            for s in range(VPC):
                sub = 2 * k + s // CVR
                idxv[n, pl.ds(s * LANES, LANES)] = (
                    ct[s % CVR] + (stripe + sub * LANES_TC)
                )
            pltpu.async_copy(
                x_hbm.at[idxv.at[n]], gv.at[pl.ds(n * CHUNK, CHUNK)], sem
            )
        return ct

    lax.fori_loop(0, TROWS_PER_W, build_fire, cts)

    def drain(n, carry):
        pltpu.make_async_copy(
            x_hbm.at[idxv.at[n]], gv.at[pl.ds(n * CHUNK, CHUNK)], sem
        ).wait()
        return carry

    lax.fori_loop(0, NCHUNKS, drain, 0)

    pltpu.sync_copy(gv, out_hbm.at[pl.ds(row0 * NCOLS, ROWS_PER_W * NCOLS)])


def kernel(x, column_indices):
    x_tiles = (
        x.reshape(BATCH // SUB, SUB, NFEAT // LANES_TC, LANES_TC)
        .swapaxes(1, 2)
        .reshape(-1)
    )
    out = _sc_gather_cols(x_tiles, column_indices)
    return out.reshape(BATCH, NCOLS)
